# baseline (speedup 1.0000x reference)
import functools

import jax
import jax.numpy as jnp
from jax import lax
from jax.experimental import pallas as pl
from jax.experimental.pallas import tpu as pltpu
from jax.experimental.pallas import tpu_sc as plsc

_NC = 2
_NS = 16
_NW = _NC * _NS
_CH = 128
_LB = 8192
_UC = 128


def _sc_gather_body(w_hbm, val_hbm, out_hbm, w_v, rows_v, gsem):
    wid = lax.axis_index("s") * _NC + lax.axis_index("c")
    k, ch = w_v.shape
    pltpu.sync_copy(w_hbm.at[wid], w_v)
    descs = []
    for j in range(k):
        rows_j = rows_v.at[pl.ds(j * ch, ch)]
        descs.append(pltpu.async_copy(val_hbm.at[w_v.at[j]], rows_j, gsem))
    for d in descs:
        d.wait()
    base = wid * (k * ch)
    pltpu.sync_copy(rows_v, out_hbm.at[pl.ds(base, k * ch)])


def _sc_gather(w3, val):
    _, k, ch = w3.shape
    b, d = val.shape
    mesh = plsc.VectorSubcoreMesh(
        core_axis_name="c", subcore_axis_name="s",
        num_cores=_NC, num_subcores=_NS)
    return pl.kernel(
        _sc_gather_body,
        out_type=jax.ShapeDtypeStruct((b, d), jnp.float32),
        mesh=mesh,
        scratch_types=[
            pltpu.VMEM((k, ch), jnp.int32),
            pltpu.VMEM((k * ch, d), jnp.float32),
            pltpu.SemaphoreType.DMA,
        ],
        compiler_params=pltpu.CompilerParams(use_tc_tiling_on_sc=False),
    )(w3, val)


def _tc_body(lo_ref, hi_ref, memT_ref, s3_ref, v3_ref, out_ref):
    i = pl.program_id(0)
    lo = lo_ref[i]
    hi = hi_ref[i]
    out_ref[...] = memT_ref[...]
    c0 = lo // _UC
    n = (hi + _UC - 1) // _UC - c0
    iota2 = lax.broadcasted_iota(jnp.int32, (_LB, _UC), 0) + i * _LB

    def chunk_body(t, carry):
        c = c0 + t
        s_row = s3_ref[pl.ds(c, 1)].reshape(1, _UC)
        v_chunk = v3_ref[pl.ds(c, 1)].reshape(64, _UC)
        eq = iota2 == jnp.broadcast_to(s_row, (_LB, _UC))
        p = eq.astype(jnp.bfloat16)
        vs_aug = jnp.concatenate(
            [v_chunk, jnp.ones((1, _UC), jnp.float32)], axis=0)
        a1 = vs_aug.astype(jnp.bfloat16)
        a2 = (vs_aug - a1.astype(jnp.float32)).astype(jnp.bfloat16)
        dn = (((1,), (1,)), ((), ()))
        u_aug = (lax.dot_general(a2, p, dn, preferred_element_type=jnp.float32)
                 + lax.dot_general(a1, p, dn,
                                   preferred_element_type=jnp.float32))
        m = u_aug[64:65, :]
        u = u_aug[0:64, :]
        out_ref[...] = jnp.where(m > 0.5, u, out_ref[...])
        return carry

    lax.fori_loop(0, n, chunk_body, 0)


def _tc_copy_scatter(memT, s3, v3, lo, hi):
    d, m = memT.shape
    nb = pl.cdiv(m, _LB)
    nch = s3.shape[0]
    grid_spec = pltpu.PrefetchScalarGridSpec(
        num_scalar_prefetch=2,
        grid=(nb,),
        in_specs=[
            pl.BlockSpec((d, _LB), lambda i, lo, hi: (0, i)),
            pl.BlockSpec((nch, 1, _UC), lambda i, lo, hi: (0, 0, 0)),
            pl.BlockSpec((nch, d, _UC), lambda i, lo, hi: (0, 0, 0)),
        ],
        out_specs=pl.BlockSpec((d, _LB), lambda i, lo, hi: (0, i)),
    )
    return pl.pallas_call(
        _tc_body,
        grid_spec=grid_spec,
        out_shape=jax.ShapeDtypeStruct((d, m), jnp.float32),
        compiler_params=pltpu.CompilerParams(
            dimension_semantics=("parallel",),
        ),
    )(lo, hi, memT, s3, v3)


def _winner_sources(idx):
    b = idx.shape[0]
    perm = jnp.argsort(idx, stable=True).astype(jnp.int32)
    s = idx[perm]
    is_last = jnp.concatenate([s[1:] != s[:-1], jnp.ones((1,), jnp.bool_)])
    pos = jnp.arange(b, dtype=jnp.int32)
    end = jnp.flip(lax.cummin(jnp.flip(jnp.where(is_last, pos, jnp.int32(b)))))
    w = perm[end]
    s_eff = jnp.where(is_last, s, jnp.int32(-1))
    return s, s_eff, w


def kernel(mem, idx, val):
    m, d = mem.shape
    b = idx.shape[0]
    k = b // (_NW * _CH)
    nch = b // _UC
    s, s_eff, w = _winner_sources(idx)
    val_s = _sc_gather(w.reshape(_NW, k, _CH), val)
    v3 = val_s.reshape(nch, _UC, d).transpose(0, 2, 1)
    s3 = s_eff.reshape(nch, 1, _UC)
    nb = pl.cdiv(m, _LB)
    bounds = jnp.arange(nb + 1, dtype=jnp.int32) * _LB
    ss = jnp.searchsorted(s, bounds, side="left").astype(jnp.int32)
    lo, hi = ss[:-1], ss[1:]
    outT = _tc_copy_scatter(mem.T, s3, v3, lo, hi)
    return outT.T


# speedup vs baseline: 1.0426x; 1.0426x over previous
import functools

import jax
import jax.numpy as jnp
from jax import lax
from jax.experimental import pallas as pl
from jax.experimental.pallas import tpu as pltpu
from jax.experimental.pallas import tpu_sc as plsc

_NC = 2
_NS = 16
_NW = _NC * _NS
_CH = 128
_LB = 6144
_UC = 128


def _sc_gather_body(w_hbm, val_hbm, out_hbm, w_v, rows_v, gsem):
    wid = lax.axis_index("s") * _NC + lax.axis_index("c")
    k, ch = w_v.shape
    pltpu.sync_copy(w_hbm.at[wid], w_v)
    descs = []
    for j in range(k):
        rows_j = rows_v.at[pl.ds(j * ch, ch)]
        descs.append(pltpu.async_copy(val_hbm.at[w_v.at[j]], rows_j, gsem))
    for d in descs:
        d.wait()
    base = wid * (k * ch)
    pltpu.sync_copy(rows_v, out_hbm.at[pl.ds(base, k * ch)])


def _sc_gather(w3, val):
    _, k, ch = w3.shape
    b, d = val.shape
    mesh = plsc.VectorSubcoreMesh(
        core_axis_name="c", subcore_axis_name="s",
        num_cores=_NC, num_subcores=_NS)
    return pl.kernel(
        _sc_gather_body,
        out_type=jax.ShapeDtypeStruct((b, d), jnp.float32),
        mesh=mesh,
        scratch_types=[
            pltpu.VMEM((k, ch), jnp.int32),
            pltpu.VMEM((k * ch, d), jnp.float32),
            pltpu.SemaphoreType.DMA,
        ],
        compiler_params=pltpu.CompilerParams(use_tc_tiling_on_sc=False),
    )(w3, val)


def _tc_body(lo_ref, hi_ref, memT_ref, s3_ref, v3_ref, out_ref):
    i = pl.program_id(0)
    lo = lo_ref[i]
    hi = hi_ref[i]
    out_ref[...] = memT_ref[...]
    c0 = lo // _UC
    n = (hi + _UC - 1) // _UC - c0
    iota2 = lax.broadcasted_iota(jnp.int32, (_LB, _UC), 0) + i * _LB

    def chunk_body(t, carry):
        c = c0 + t
        s_row = s3_ref[pl.ds(c, 1)].reshape(1, _UC)
        v_chunk = v3_ref[pl.ds(c, 1)].reshape(64, _UC)
        eq = iota2 == jnp.broadcast_to(s_row, (_LB, _UC))
        p = eq.astype(jnp.bfloat16)
        vs_aug = jnp.concatenate(
            [v_chunk, jnp.ones((1, _UC), jnp.float32)], axis=0)
        a1 = vs_aug.astype(jnp.bfloat16)
        a2 = (vs_aug - a1.astype(jnp.float32)).astype(jnp.bfloat16)
        dn = (((1,), (1,)), ((), ()))
        u_aug = (lax.dot_general(a2, p, dn, preferred_element_type=jnp.float32)
                 + lax.dot_general(a1, p, dn,
                                   preferred_element_type=jnp.float32))
        m = u_aug[64:65, :]
        u = u_aug[0:64, :]
        out_ref[...] = jnp.where(m > 0.5, u, out_ref[...])
        return carry

    lax.fori_loop(0, n, chunk_body, 0)


def _tc_copy_scatter(memT, s3, v3, lo, hi):
    d, m = memT.shape
    nb = pl.cdiv(m, _LB)
    nch = s3.shape[0]
    grid_spec = pltpu.PrefetchScalarGridSpec(
        num_scalar_prefetch=2,
        grid=(nb,),
        in_specs=[
            pl.BlockSpec((d, _LB), lambda i, lo, hi: (0, i)),
            pl.BlockSpec((nch, 1, _UC), lambda i, lo, hi: (0, 0, 0)),
            pl.BlockSpec((nch, d, _UC), lambda i, lo, hi: (0, 0, 0)),
        ],
        out_specs=pl.BlockSpec((d, _LB), lambda i, lo, hi: (0, i)),
    )
    return pl.pallas_call(
        _tc_body,
        grid_spec=grid_spec,
        out_shape=jax.ShapeDtypeStruct((d, m), jnp.float32),
        compiler_params=pltpu.CompilerParams(
            dimension_semantics=("parallel",),
        ),
    )(lo, hi, memT, s3, v3)


def _winner_sources(idx):
    b = idx.shape[0]
    perm = jnp.argsort(idx, stable=True).astype(jnp.int32)
    s = idx[perm]
    is_last = jnp.concatenate([s[1:] != s[:-1], jnp.ones((1,), jnp.bool_)])
    pos = jnp.arange(b, dtype=jnp.int32)
    end = jnp.flip(lax.cummin(jnp.flip(jnp.where(is_last, pos, jnp.int32(b)))))
    w = perm[end]
    s_eff = jnp.where(is_last, s, jnp.int32(-1))
    return s, s_eff, w


def kernel(mem, idx, val):
    m, d = mem.shape
    b = idx.shape[0]
    k = b // (_NW * _CH)
    nch = b // _UC
    s, s_eff, w = _winner_sources(idx)
    val_s = _sc_gather(w.reshape(_NW, k, _CH), val)
    v3 = val_s.reshape(nch, _UC, d).transpose(0, 2, 1)
    s3 = s_eff.reshape(nch, 1, _UC)
    nb = pl.cdiv(m, _LB)
    bounds = jnp.arange(nb + 1, dtype=jnp.int32) * _LB
    ss = jnp.searchsorted(s, bounds, side="left").astype(jnp.int32)
    lo, hi = ss[:-1], ss[1:]
    outT = _tc_copy_scatter(mem.T, s3, v3, lo, hi)
    return outT.T
